# Initial kernel scaffold; baseline (speedup 1.0000x reference)
import jax
import jax.numpy as jnp
from jax.experimental import pallas as pl


def kernel(queries, keys, values, clusters):
    raise NotImplementedError("write your pallas kernel here")


# speedup vs baseline: 9.7220x; 9.7220x over previous
import jax
import jax.numpy as jnp
from jax.experimental import pallas as pl
from jax.experimental.pallas import tpu as pltpu

B, N, D = 32, 8192, 64
C = 129


def _attn_body(q_ref, k_ref, v_ref, cl_ref, out_ref, acol_ref):
    cb = cl_ref[0, 0, :]
    iota = jax.lax.broadcasted_iota(jnp.int32, (C, N), 0)
    onehot = (cb[None, :] == iota).astype(jnp.float32)

    counts = jnp.sum(onehot, axis=1)
    inv = 1.0 / counts

    qb = q_ref[0]
    kb = k_ref[0]
    vb = v_ref[0]

    segq = jnp.dot(onehot, qb, preferred_element_type=jnp.float32)
    segk = jnp.dot(onehot, kb, preferred_element_type=jnp.float32)
    segv = jnp.dot(onehot, vb, preferred_element_type=jnp.float32)

    qc = segq * inv[:, None]
    kc = segk * inv[:, None]
    vc = segv * inv[:, None]

    qk = jax.lax.dot_general(qc, kc, (((1,), (1,)), ((), ())),
                             preferred_element_type=jnp.float32)
    a = jax.nn.softmax(qk, axis=-1)
    aw = a * counts[None, :]
    aw = aw / jnp.sum(aw, axis=-1, keepdims=True)

    v2 = jnp.dot(aw, vc, preferred_element_type=jnp.float32)

    out_ref[0] = jax.lax.dot_general(onehot, v2, (((0,), (0,)), ((), ())),
                                     preferred_element_type=jnp.float32)

    col0 = (jax.lax.broadcasted_iota(jnp.int32, (C, C), 1) == 0).astype(jnp.float32)
    acol_ref[0, 0, :] = jnp.sum(aw * col0, axis=1)


def kernel(queries, keys, values, clusters):
    cl3 = clusters.reshape(B, 1, N)
    out, acol = pl.pallas_call(
        _attn_body,
        grid=(B,),
        in_specs=[
            pl.BlockSpec((1, N, D), lambda b: (b, 0, 0)),
            pl.BlockSpec((1, N, D), lambda b: (b, 0, 0)),
            pl.BlockSpec((1, N, D), lambda b: (b, 0, 0)),
            pl.BlockSpec((1, 1, N), lambda b: (b, 0, 0)),
        ],
        out_specs=[
            pl.BlockSpec((1, N, D), lambda b: (b, 0, 0)),
            pl.BlockSpec((1, 1, C), lambda b: (b, 0, 0)),
        ],
        out_shape=[
            jax.ShapeDtypeStruct((B, N, D), jnp.float32),
            jax.ShapeDtypeStruct((B, 1, C), jnp.float32),
        ],
    )(queries, keys, values, cl3)
    return (out, acol.reshape(B, C))
